# baseline (speedup 1.0000x reference)
import functools

import jax
import jax.numpy as jnp
from jax import lax
from jax.experimental import pallas as pl
from jax.experimental.pallas import tpu as pltpu
from jax.experimental.pallas import tpu_sc as plsc

NUM_EMB = 1000000
DIM = 64
BATCH = 16384
SEQ = 50
NC, NS = 2, 16
NW = NC * NS
CHUNK = 128
NBLK = SEQ * (BATCH // CHUNK)
BLK_PER_W = NBLK // NW
IDS_PER_W = BLK_PER_W * CHUNK
JB = BATCH // CHUNK

_mesh = plsc.VectorSubcoreMesh(core_axis_name="c", subcore_axis_name="s")


@functools.partial(
    pl.kernel,
    out_type=jax.ShapeDtypeStruct((SEQ, DIM, BATCH), jnp.float32),
    mesh=_mesh,
    scratch_types=[
        pltpu.VMEM((IDS_PER_W,), jnp.int32),
        pltpu.VMEM((CHUNK,), jnp.int32),
        pltpu.VMEM((CHUNK,), jnp.int32),
        pltpu.VMEM((CHUNK,), jnp.int32),
        pltpu.VMEM((CHUNK, 128), jnp.float32),
        pltpu.VMEM((CHUNK, 128), jnp.float32),
        pltpu.VMEM((DIM, CHUNK + 1), jnp.float32),
        pltpu.VMEM((DIM, CHUNK + 1), jnp.float32),
        pltpu.SemaphoreType.DMA,
        pltpu.SemaphoreType.DMA,
        pltpu.SemaphoreType.DMA,
        pltpu.SemaphoreType.DMA,
    ],
    compiler_params=pltpu.CompilerParams(
        use_tc_tiling_on_sc=True, needs_layout_passes=False
    ),
)
def _gather_kernel(table_hbm, idx_hbm, out_hbm,
                   ids_all, kidx, idx2_0, idx2_1, rows_0, rows_1,
                   obuf_0, obuf_1, gsem_0, gsem_1, ssem_0, ssem_1):
    wid = lax.axis_index("s") * NC + lax.axis_index("c")
    base = wid * BLK_PER_W
    idx2_b = (idx2_0, idx2_1)
    rows_b = (rows_0, rows_1)
    obuf_b = (obuf_0, obuf_1)
    gsem = (gsem_0, gsem_1)
    ssem = (ssem_0, ssem_1)

    lane = lax.broadcasted_iota(jnp.int32, (16,), 0)
    for b0 in range(CHUNK // 16):
        kidx[pl.ds(b0 * 16, 16)] = lane + b0 * 16

    pltpu.sync_copy(idx_hbm.at[pl.ds(base * CHUNK, IDS_PER_W)], ids_all)

    def fire(t, h):
        off = t * CHUNK
        for b0 in range(CHUNK // 16):
            v = ids_all[pl.ds(off + b0 * 16, 16)]
            idx2_b[h][pl.ds(b0 * 16, 16)] = lax.shift_right_logical(v, 1)
        pltpu.make_async_copy(table_hbm.at[idx2_b[h]], rows_b[h], gsem[h]).start()

    def drain_gather(h):
        pltpu.make_async_copy(table_hbm.at[idx2_b[h]], rows_b[h], gsem[h]).wait()

    def transpose(t, h):
        off = t * CHUNK
        c_rows = [lane + c0 for c0 in range(0, DIM, 16)]
        for b0 in range(CHUNK // 16):
            idsv = ids_all[pl.ds(off + b0 * 16, 16)]
            pcolv = lax.shift_left(lax.bitwise_and(idsv, 1), 6)
            kv = kidx[pl.ds(b0 * 16, 16)]
            pcols = [pcolv[j] for j in range(16)]
            cols = [lax.broadcast(kv[j], (16,)) for j in range(16)]
            for j in range(16):
                k = b0 * 16 + j
                vals = [
                    rows_b[h][k, pl.ds(pcols[j] + q * 16, 16)]
                    for q in range(DIM // 16)
                ]
                for q in range(DIM // 16):
                    plsc.store_scatter(obuf_b[h], [c_rows[q], cols[j]], vals[q])

    def store_copy(t, h):
        g = base + t
        s = g // JB
        jb = g - s * JB
        return pltpu.make_async_copy(
            obuf_b[h].at[:, pl.ds(0, CHUNK)],
            out_hbm.at[s, :, pl.ds(jb * CHUNK, CHUNK)],
            ssem[h],
        )

    fire(0, 0)
    fire(1, 1)

    def body(p, carry):
        t0 = 2 * p
        for h in (0, 1):
            t = t0 + h
            drain_gather(h)

            @pl.when(t >= 2)
            def _():
                store_copy(t - 2, h).wait()

            transpose(t, h)
            store_copy(t, h).start()

            @pl.when(t + 2 < BLK_PER_W)
            def _():
                fire(t + 2, h)

        return carry

    lax.fori_loop(0, BLK_PER_W // 2, body, 0)
    store_copy(BLK_PER_W - 2, 0).wait()
    store_copy(BLK_PER_W - 1, 1).wait()


def kernel(input_ids, weight):
    table = weight.reshape(NUM_EMB // 2, 128)
    idx_flat = input_ids.T.reshape(-1)
    out3 = _gather_kernel(table, idx_flat)
    return jnp.transpose(out3, (2, 0, 1))


# speedup vs baseline: 1.3207x; 1.3207x over previous
import functools

import jax
import jax.numpy as jnp
from jax import lax
from jax.experimental import pallas as pl
from jax.experimental.pallas import tpu as pltpu
from jax.experimental.pallas import tpu_sc as plsc

NUM_EMB = 1000000
DIM = 64
BATCH = 16384
SEQ = 50
B_TOTAL = BATCH * SEQ
NC, NS = 2, 16
NW = NC * NS
B_PER_W = B_TOTAL // NW
CHUNK = 128
K = 4
GROUP = K * CHUNK
G = B_PER_W // GROUP

_mesh = plsc.VectorSubcoreMesh(core_axis_name="c", subcore_axis_name="s")


@functools.partial(
    pl.kernel,
    out_type=jax.ShapeDtypeStruct((B_TOTAL, DIM), jnp.float32),
    mesh=_mesh,
    scratch_types=[
        pltpu.VMEM((B_PER_W,), jnp.int32),
        pltpu.VMEM((2, GROUP, DIM), jnp.float32),
        pltpu.SemaphoreType.DMA,
        pltpu.SemaphoreType.DMA,
    ],
    compiler_params=pltpu.CompilerParams(use_tc_tiling_on_sc=False),
)
def _gather_kernel(table_hbm, idx_hbm, out_hbm, idx_v, rows_v, sem0, sem1):
    wid = lax.axis_index("s") * NC + lax.axis_index("c")
    base = wid * B_PER_W
    pltpu.sync_copy(idx_hbm.at[pl.ds(base, B_PER_W)], idx_v)
    sems = (sem0, sem1)

    def handle(g, half, j):
        src = table_hbm.at[idx_v.at[pl.ds((g * K + j) * CHUNK, CHUNK)]]
        dst = rows_v.at[half, pl.ds(j * CHUNK, CHUNK)]
        return pltpu.make_async_copy(src, dst, sems[half])

    def fire(g, half):
        for j in range(K):
            handle(g, half, j).start()

    def drain(g, half):
        for j in range(K):
            handle(g, half, j).wait()

    def store(g, half):
        pltpu.sync_copy(rows_v.at[half], out_hbm.at[pl.ds(base + g * GROUP, GROUP)])

    fire(0, 0)

    def body(t, carry):
        g0 = 2 * t
        fire(g0 + 1, 1)
        drain(g0, 0)
        store(g0, 0)
        fire(g0 + 2, 0)
        drain(g0 + 1, 1)
        store(g0 + 1, 1)
        return carry

    lax.fori_loop(0, G // 2 - 1, body, 0)
    fire(G - 1, 1)
    drain(G - 2, 0)
    store(G - 2, 0)
    drain(G - 1, 1)
    store(G - 1, 1)


def kernel(input_ids, weight):
    idx_flat = input_ids.reshape(-1).astype(jnp.int32)
    out = _gather_kernel(weight, idx_flat)
    return out.reshape(BATCH, SEQ, DIM)
